# baseline (speedup 1.0000x reference)
import dataclasses
import functools

import jax
import jax.numpy as jnp
from jax import lax
from jax.experimental import pallas as pl
from jax.experimental.pallas import tpu as pltpu
from jax.experimental.pallas import tpu_sc as plsc

_NB = 6
_ROWS = 4096
_COLS = 6144
_SYM = _COLS // _NB


def _sc_body(bits_hbm, tbl_hbm, ore_hbm, oim_hbm, tbl_v):
    pltpu.sync_copy(tbl_hbm, tbl_v)
    lanes = jnp.arange(16, dtype=jnp.int32)
    zero = jnp.zeros((16,), jnp.int32)

    def body(bits_vmem, ore_vmem, oim_vmem):
        @pl.loop(0, _SYM // 16)
        def _(g):
            col = g * (16 * _NB)
            idx = zero
            for k in range(_NB):
                bk = plsc.load_gather(bits_vmem, [zero, col + lanes * _NB + k])
                idx = idx * 2 + bk
            ore_vmem[0, pl.ds(g * 16, 16)] = plsc.load_gather(tbl_v, [idx])
            oim_vmem[0, pl.ds(g * 16, 16)] = plsc.load_gather(tbl_v, [idx + 64])

    pltpu.emit_pipeline(
        body,
        grid=(_ROWS,),
        in_specs=[pl.BlockSpec((1, _COLS), lambda i: (i, 0))],
        out_specs=[
            pl.BlockSpec((1, _SYM), lambda i: (i, 0)),
            pl.BlockSpec((1, _SYM), lambda i: (i, 0)),
        ],
        core_axis_name=("c", "s"),
        dimension_semantics=(pltpu.PARALLEL,),
    )(bits_hbm, ore_hbm, oim_hbm)


@jax.jit
def kernel(inputs, points):
    tbl = jnp.concatenate(
        [jnp.real(points), jnp.imag(points)]).astype(jnp.float32)
    mesh = plsc.VectorSubcoreMesh(core_axis_name="c", subcore_axis_name="s")
    cp = pltpu.CompilerParams()
    if "needs_layout_passes" in pltpu.CompilerParams.__dataclass_fields__:
        cp = dataclasses.replace(cp, needs_layout_passes=False)
    kfn = pl.kernel(
        _sc_body,
        out_type=[
            jax.ShapeDtypeStruct((_ROWS, _SYM), jnp.float32),
            jax.ShapeDtypeStruct((_ROWS, _SYM), jnp.float32),
        ],
        mesh=mesh,
        scratch_types=[pltpu.VMEM((128,), jnp.float32)],
        compiler_params=cp,
    )
    ore, oim = kfn(inputs, tbl)
    return jax.lax.complex(ore, oim)


# speedup vs baseline: 1.0036x; 1.0036x over previous
import dataclasses
import functools

import jax
import jax.numpy as jnp
from jax import lax
from jax.experimental import pallas as pl
from jax.experimental.pallas import tpu as pltpu
from jax.experimental.pallas import tpu_sc as plsc

_NB = 6
_ROWS = 4096
_COLS = 6144
_SYM = _COLS // _NB


_BR = 4
_UNROLL = 4


def _sc_body(bits_hbm, tbl_hbm, ore_hbm, oim_hbm, tbl_v):
    pltpu.sync_copy(tbl_hbm, tbl_v)
    lanes = jnp.arange(16, dtype=jnp.int32)
    bases = [lanes * _NB + k for k in range(_NB)]
    zero = jnp.zeros((16,), jnp.int32)

    def body(bits_vmem, ore_vmem, oim_vmem):
        @pl.loop(0, _SYM // (16 * _UNROLL))
        def _(g):
            for r in range(_BR):
                row = jnp.full((16,), r, jnp.int32)
                for q in range(_UNROLL):
                    col = (g * _UNROLL + q) * (16 * _NB)
                    idx = zero
                    for k in range(_NB):
                        idx = idx * 2 + plsc.load_gather(
                            bits_vmem, [row, col + bases[k]])
                    s = pl.ds(g * (16 * _UNROLL) + q * 16, 16)
                    ore_vmem[r, s] = plsc.load_gather(tbl_v, [idx])
                    oim_vmem[r, s] = plsc.load_gather(tbl_v, [idx + 64])

    pltpu.emit_pipeline(
        body,
        grid=(_ROWS // _BR,),
        in_specs=[pl.BlockSpec((_BR, _COLS), lambda i: (i, 0))],
        out_specs=[
            pl.BlockSpec((_BR, _SYM), lambda i: (i, 0)),
            pl.BlockSpec((_BR, _SYM), lambda i: (i, 0)),
        ],
        core_axis_name=("c", "s"),
        dimension_semantics=(pltpu.PARALLEL,),
    )(bits_hbm, ore_hbm, oim_hbm)


@jax.jit
def kernel(inputs, points):
    tbl = jnp.concatenate(
        [jnp.real(points), jnp.imag(points)]).astype(jnp.float32)
    mesh = plsc.VectorSubcoreMesh(core_axis_name="c", subcore_axis_name="s")
    cp = pltpu.CompilerParams()
    if "needs_layout_passes" in pltpu.CompilerParams.__dataclass_fields__:
        cp = dataclasses.replace(cp, needs_layout_passes=False)
    kfn = pl.kernel(
        _sc_body,
        out_type=[
            jax.ShapeDtypeStruct((_ROWS, _SYM), jnp.float32),
            jax.ShapeDtypeStruct((_ROWS, _SYM), jnp.float32),
        ],
        mesh=mesh,
        scratch_types=[pltpu.VMEM((128,), jnp.float32)],
        compiler_params=cp,
    )
    ore, oim = kfn(inputs, tbl)
    return jax.lax.complex(ore, oim)


# speedup vs baseline: 1.4166x; 1.4116x over previous
import functools

import jax
import jax.numpy as jnp
import numpy as np
from jax.experimental import pallas as pl
from jax.experimental.pallas import tpu as pltpu

_NB = 6
_NPTS = 64
_ROWS = 4096
_COLS = 6144
_SYM = _COLS // _NB
_TILE_IN = 128 * _NB
_BLOCK_R = 512


def _weight_tile() -> np.ndarray:
    w = np.zeros((_TILE_IN, 128), np.float32)
    for s in range(128):
        for k in range(_NB):
            w[s * _NB + k, s] = float(2 ** (_NB - 1 - k))
    return w


def _body(bits_ref, w_ref, pre_ref, pim_ref, ore_ref, oim_ref):
    w = w_ref[...]
    pre = jnp.broadcast_to(pre_ref[...], (_BLOCK_R, _NPTS))
    pim = jnp.broadcast_to(pim_ref[...], (_BLOCK_R, _NPTS))
    for t in range(_SYM // 128):
        seg = bits_ref[:, t * _TILE_IN:(t + 1) * _TILE_IN].astype(jnp.bfloat16)
        idxf = jnp.dot(seg, w, preferred_element_type=jnp.float32)
        idx = idxf.astype(jnp.int32)
        ore_ref[:, t * 128:(t + 1) * 128] = jnp.take_along_axis(
            pre, idx, axis=1, mode="promise_in_bounds")
        oim_ref[:, t * 128:(t + 1) * 128] = jnp.take_along_axis(
            pim, idx, axis=1, mode="promise_in_bounds")


@jax.jit
def kernel(inputs, points):
    pre = jnp.real(points).astype(jnp.float32)
    pim = jnp.imag(points).astype(jnp.float32)
    w = jnp.asarray(_weight_tile(), dtype=jnp.bfloat16)
    grid = (_ROWS // _BLOCK_R,)
    out_shape = [
        jax.ShapeDtypeStruct((_ROWS, _SYM), jnp.float32),
        jax.ShapeDtypeStruct((_ROWS, _SYM), jnp.float32),
    ]
    ore, oim = pl.pallas_call(
        _body,
        grid=grid,
        in_specs=[
            pl.BlockSpec((_BLOCK_R, _COLS), lambda i: (i, 0)),
            pl.BlockSpec((_TILE_IN, 128), lambda i: (0, 0)),
            pl.BlockSpec((_NPTS,), lambda i: (0,)),
            pl.BlockSpec((_NPTS,), lambda i: (0,)),
        ],
        out_specs=[
            pl.BlockSpec((_BLOCK_R, _SYM), lambda i: (i, 0)),
            pl.BlockSpec((_BLOCK_R, _SYM), lambda i: (i, 0)),
        ],
        out_shape=out_shape,
    )(inputs, w, pre, pim)
    return jax.lax.complex(ore, oim)
